# baseline (speedup 1.0000x reference)
import functools

import jax
import jax.numpy as jnp
from jax import lax
from jax.experimental import pallas as pl
from jax.experimental.pallas import tpu as pltpu
from jax.experimental.pallas import tpu_sc as plsc

D = 1024
H = 4096
E = 8
N_TOK = 2048
N_DISP = N_TOK * 2

NC = 2
NS = 16
NW = NC * NS
CHUNK = N_DISP // NW
L = 16

TM = 256
TH = 1024
NMD = N_DISP // TM
NHD = H // TH
S_STEPS = NMD + E - 1


def _router_body(x_ref, wr_ref, te_ref, tw_ref, lbl_ref, zl_ref):
    x = x_ref[...]
    wr = wr_ref[...]
    logits = lax.dot_general(x, wr, (((1,), (1,)), ((), ())),
                             preferred_element_type=jnp.float32)
    mx = jnp.max(logits, axis=-1, keepdims=True)
    ex = jnp.exp(logits - mx)
    denom = jnp.sum(ex, axis=-1, keepdims=True)
    probs = ex / denom
    col = lax.broadcasted_iota(jnp.int32, probs.shape, 1)
    m1 = jnp.max(probs, axis=-1, keepdims=True)
    i1 = jnp.min(jnp.where(probs == m1, col, E), axis=-1, keepdims=True)
    oh1 = (col == i1)
    p2 = jnp.where(oh1, -1.0, probs)
    m2 = jnp.max(p2, axis=-1, keepdims=True)
    i2 = jnp.min(jnp.where(p2 == m2, col, E), axis=-1, keepdims=True)
    oh2 = (col == i2)
    wsum = m1 + m2
    te_ref[...] = jnp.concatenate([i1, i2], axis=1)
    tw_ref[...] = jnp.concatenate([m1 / wsum, m2 / wsum], axis=1)
    lse = jnp.log(denom) + mx
    zl_ref[...] = jnp.sum(jnp.square(lse), axis=0, keepdims=True) / N_TOK
    em = oh1.astype(jnp.float32) + oh2.astype(jnp.float32)
    f = jnp.sum(em, axis=0, keepdims=True) / (N_TOK * 2.0)
    P = jnp.sum(probs, axis=0, keepdims=True) / N_TOK
    lbl_ref[...] = E * jnp.sum(f * P, axis=1, keepdims=True)


def _dispatch_body(te_hbm, tw_hbm, x_hbm,
                   pos_hbm, xs_hbm, ws_hbm, cnt_hbm,
                   e_all, pos_chunk, pos_half, wvals, wrow, idx_tok, xbuf,
                   cnt_buf, run_buf, sem):
    wid = lax.axis_index("s") * NC + lax.axis_index("c")
    base_p = wid * CHUNK
    my_first_vreg = base_p // L

    pltpu.sync_copy(te_hbm, e_all)

    lane = lax.iota(jnp.int32, L)
    zero_v = jnp.zeros((L,), jnp.int32)

    def count_step(i, carry):
        total, before = carry
        v = e_all[pl.ds(i * L, L)]
        hist = zero_v
        for e in range(E):
            pc = jnp.sum((v == e).astype(jnp.int32))
            hist = hist + jnp.where(lane == e, pc, 0)
        total = total + hist
        before = before + jnp.where(i < my_first_vreg, hist, zero_v)
        return total, before

    total, before = lax.fori_loop(0, N_DISP // L, count_step,
                                  (zero_v, zero_v))

    @pl.when(wid == 0)
    def _():
        cnt_buf[...] = total
        pltpu.sync_copy(cnt_buf, cnt_hbm)

    off_excl = plsc.cumsum(total) - total
    run_buf[...] = off_excl + before

    for i in range(CHUNK // L):
        v = e_all[pl.ds((my_first_vreg + i) * L, L)]
        start_here = plsc.load_gather(run_buf, [v])
        posv = zero_v
        hist = zero_v
        for e in range(E):
            m = (v == e)
            c = plsc.cumsum(m.astype(jnp.int32))
            posv = posv + jnp.where(m, c - 1, 0)
            hist = hist + jnp.where(lane == e, c[L - 1], 0)
        pos_chunk[pl.ds(i * L, L)] = start_here + posv
        run_buf[...] = run_buf[...] + hist

    pltpu.sync_copy(pos_chunk, pos_hbm.at[pl.ds(base_p, CHUNK)])

    pltpu.sync_copy(tw_hbm.at[pl.ds(base_p, CHUNK)], wvals)
    for c in range(CHUNK // L):
        wv = wvals[pl.ds(c * L, L)]
        r = c * L + lane
        plsc.store_scatter(wrow, [r, zero_v], wv)
    pltpu.async_copy(wrow, ws_hbm.at[pos_chunk], sem).wait()

    for half in range(2):
        hbase = base_p + half * (CHUNK // 2)
        for c in range(CHUNK // 2 // L):
            tok = lax.shift_right_logical(hbase + c * L + lane, 1)
            idx_tok[pl.ds(c * L, L)] = tok
            pos_half[pl.ds(c * L, L)] = pos_chunk[pl.ds(half * (CHUNK // 2)
                                                        + c * L, L)]
        pltpu.async_copy(x_hbm.at[idx_tok], xbuf, sem).wait()
        pltpu.async_copy(xbuf, xs_hbm.at[pos_half], sem).wait()


def _gmm_body(se_ref, st_ref, slo_ref, shi_ref, sfirst_ref,
              xs_ref, ws_ref, wsu_ref, wru_ref, wsd_ref, wrd_ref, out_ref):
    h_idx = pl.program_id(0)
    s_idx = pl.program_id(1)

    @pl.when((h_idx == 0) & (s_idx == 0))
    def _():
        out_ref[...] = jnp.zeros_like(out_ref)

    t = st_ref[s_idx]
    lo = slo_ref[s_idx]
    hi = shi_ref[s_idx]
    base = t * TM
    row = base + lax.broadcasted_iota(jnp.int32, (TM, 1), 0)
    msk = (row >= lo) & (row < hi)
    xm = xs_ref[...]
    we = (wsu_ref[...] + wru_ref[0]).astype(jnp.bfloat16)
    hblk = jnp.dot(xm.astype(jnp.bfloat16), we,
                   preferred_element_type=jnp.float32)
    a = jnp.square(jnp.maximum(hblk, 0.0))
    wcol = jnp.where(msk, ws_ref[pl.ds(base, TM), 0:1], 0.0)
    aw = (a * wcol).astype(jnp.bfloat16)
    wd = (wsd_ref[...] + wrd_ref[0]).astype(jnp.bfloat16)
    y = lax.dot_general(aw, wd, (((1,), (1,)), ((), ())),
                        preferred_element_type=jnp.float32)
    cur = out_ref[pl.ds(base, TM), :]
    out_ref[pl.ds(base, TM), :] = cur + y


def _combine_body(pos_hbm, ysw_hbm, out_hbm,
                  posbuf, idx_even, idx_odd, ybuf, ybuf2, sem):
    wid = lax.axis_index("s") * NC + lax.axis_index("c")
    base_t = wid * (N_TOK // NW)
    lane = lax.iota(jnp.int32, L)

    pltpu.sync_copy(pos_hbm.at[pl.ds(base_t * 2, CHUNK)], posbuf)

    for it in range(2):
        for c in range(2):
            sbase = it * 64 + c * 2 * L
            ev = plsc.load_gather(posbuf, [sbase + 2 * lane])
            od = plsc.load_gather(posbuf, [sbase + 2 * lane + 1])
            idx_even[pl.ds(c * L, L)] = ev
            idx_odd[pl.ds(c * L, L)] = od
        pltpu.async_copy(ysw_hbm.at[idx_even], ybuf, sem).wait()
        pltpu.async_copy(ysw_hbm.at[idx_odd], ybuf2, sem).wait()

        def add_step(i, _):
            j = i // (D // L)
            cc = i % (D // L)
            ybuf[j, pl.ds(cc * L, L)] = (ybuf[j, pl.ds(cc * L, L)]
                                         + ybuf2[j, pl.ds(cc * L, L)])
            return 0

        lax.fori_loop(0, 32 * (D // L), add_step, 0)
        pltpu.sync_copy(ybuf, out_hbm.at[pl.ds(base_t + it * 32, 32)])


def _step_metadata(cnt):
    cnt = cnt[:E]
    off = jnp.concatenate([jnp.zeros((1,), jnp.int32),
                           jnp.cumsum(cnt, dtype=jnp.int32)])
    first_t = off[:E] // TM
    last_t = (off[1:] + TM - 1) // TM - 1
    n_e = jnp.where(cnt > 0, last_t - first_t + 1, 0)
    estart = jnp.concatenate([jnp.zeros((1,), jnp.int32),
                              jnp.cumsum(n_e, dtype=jnp.int32)])
    total = estart[E]
    s = jnp.arange(S_STEPS, dtype=jnp.int32)
    e_s = jnp.sum((s[:, None] >= estart[1:][None, :]).astype(jnp.int32),
                  axis=1)
    valid = s < total
    e_c = jnp.minimum(e_s, E - 1)
    tile = first_t[e_c] + (s - estart[e_c])
    tile = jnp.where(valid, tile, NMD - 1).astype(jnp.int32)
    lo = jnp.where(valid, jnp.maximum(off[e_c], tile * TM), 0)
    hi = jnp.where(valid, jnp.minimum(off[e_c + 1], tile * TM + TM), 0)
    tile_prev = jnp.concatenate([jnp.full((1,), -1, jnp.int32), tile[:-1]])
    sfirst = (valid & (tile != tile_prev)).astype(jnp.int32)
    return (e_c.astype(jnp.int32), tile, lo.astype(jnp.int32),
            hi.astype(jnp.int32), sfirst)


def kernel(x, W_router, W_shared_up, W_shared_down, W_routed_up, W_routed_down):
    B, T, Dx = x.shape
    N = B * T
    xf = x.reshape(N, Dx)

    te, tw, lbl, zl = pl.pallas_call(
        _router_body,
        out_shape=[
            jax.ShapeDtypeStruct((N, 2), jnp.int32),
            jax.ShapeDtypeStruct((N, 2), jnp.float32),
            jax.ShapeDtypeStruct((1, 1), jnp.float32),
            jax.ShapeDtypeStruct((1, 1), jnp.float32),
        ],
    )(xf, W_router)

    te_flat = te.reshape(N_DISP)
    tw_flat = tw.reshape(N_DISP)

    mesh = plsc.VectorSubcoreMesh(core_axis_name="c", subcore_axis_name="s",
                                  num_cores=NC)
    dispatch = functools.partial(
        pl.kernel,
        out_type=[
            jax.ShapeDtypeStruct((N_DISP,), jnp.int32),
            jax.ShapeDtypeStruct((N_DISP, D), jnp.float32),
            jax.ShapeDtypeStruct((N_DISP, 128), jnp.float32),
            jax.ShapeDtypeStruct((L,), jnp.int32),
        ],
        mesh=mesh,
        compiler_params=pltpu.CompilerParams(needs_layout_passes=False),
        scratch_types=[
            pltpu.VMEM((N_DISP,), jnp.int32),
            pltpu.VMEM((CHUNK,), jnp.int32),
            pltpu.VMEM((CHUNK // 2,), jnp.int32),
            pltpu.VMEM((CHUNK,), jnp.float32),
            pltpu.VMEM((CHUNK, 128), jnp.float32),
            pltpu.VMEM((CHUNK // 2,), jnp.int32),
            pltpu.VMEM((CHUNK // 2, D), jnp.float32),
            pltpu.VMEM((L,), jnp.int32),
            pltpu.VMEM((L,), jnp.int32),
            pltpu.SemaphoreType.DMA,
        ],
    )(_dispatch_body)
    pos, xs, ws, cnt = dispatch(te_flat, tw_flat, xf)

    se, st, slo, shi, sfirst = _step_metadata(cnt)

    ysw = pl.pallas_call(
        _gmm_body,
        grid_spec=pltpu.PrefetchScalarGridSpec(
            num_scalar_prefetch=5,
            grid=(NHD, S_STEPS),
            in_specs=[
                pl.BlockSpec((TM, D),
                             lambda h, s, se_, st_, *_: (st_[s], 0)),
                pl.BlockSpec((N_DISP, 128), lambda h, s, *_: (0, 0)),
                pl.BlockSpec((D, TH), lambda h, s, *_: (0, h)),
                pl.BlockSpec((1, D, TH),
                             lambda h, s, se_, *_: (se_[s], 0, h)),
                pl.BlockSpec((D, TH), lambda h, s, *_: (0, h)),
                pl.BlockSpec((1, D, TH),
                             lambda h, s, se_, *_: (se_[s], 0, h)),
            ],
            out_specs=pl.BlockSpec((N_DISP, D), lambda h, s, *_: (0, 0)),
        ),
        out_shape=jax.ShapeDtypeStruct((N_DISP, D), jnp.float32),
    )(se, st, slo, shi, sfirst, xs, ws, W_shared_up, W_routed_up,
      W_shared_down, W_routed_down)

    combine = functools.partial(
        pl.kernel,
        out_type=jax.ShapeDtypeStruct((N, D), jnp.float32),
        mesh=mesh,
        compiler_params=pltpu.CompilerParams(needs_layout_passes=False),
        scratch_types=[
            pltpu.VMEM((CHUNK,), jnp.int32),
            pltpu.VMEM((2 * L,), jnp.int32),
            pltpu.VMEM((2 * L,), jnp.int32),
            pltpu.VMEM((32, D), jnp.float32),
            pltpu.VMEM((32, D), jnp.float32),
            pltpu.SemaphoreType.DMA,
        ],
    )(_combine_body)
    out = combine(pos, ysw)

    return out.reshape(B, T, Dx), lbl.reshape(()), zl.reshape(())


# speedup vs baseline: 1.0115x; 1.0115x over previous
import functools

import jax
import jax.numpy as jnp
from jax import lax
from jax.experimental import pallas as pl
from jax.experimental.pallas import tpu as pltpu
from jax.experimental.pallas import tpu_sc as plsc

D = 1024
H = 4096
E = 8
N_TOK = 2048
N_DISP = N_TOK * 2

NC = 2
NS = 16
NW = NC * NS
CHUNK = N_DISP // NW
L = 16

TM = 256
TH = 1024
NMD = N_DISP // TM
NHD = H // TH
S_STEPS = NMD + E - 1


def _router_body(x_ref, wr_ref, te_ref, tw_ref, lbl_ref, zl_ref):
    x = x_ref[...]
    wr = wr_ref[...]
    logits = lax.dot_general(x, wr, (((1,), (1,)), ((), ())),
                             preferred_element_type=jnp.float32)
    mx = jnp.max(logits, axis=-1, keepdims=True)
    ex = jnp.exp(logits - mx)
    denom = jnp.sum(ex, axis=-1, keepdims=True)
    probs = ex / denom
    col = lax.broadcasted_iota(jnp.int32, probs.shape, 1)
    m1 = jnp.max(probs, axis=-1, keepdims=True)
    i1 = jnp.min(jnp.where(probs == m1, col, E), axis=-1, keepdims=True)
    oh1 = (col == i1)
    p2 = jnp.where(oh1, -1.0, probs)
    m2 = jnp.max(p2, axis=-1, keepdims=True)
    i2 = jnp.min(jnp.where(p2 == m2, col, E), axis=-1, keepdims=True)
    oh2 = (col == i2)
    wsum = m1 + m2
    te_ref[...] = jnp.concatenate([i1, i2], axis=1)
    tw_ref[...] = jnp.concatenate([m1 / wsum, m2 / wsum], axis=1)
    lse = jnp.log(denom) + mx
    zl_ref[...] = jnp.sum(jnp.square(lse), axis=0, keepdims=True) / N_TOK
    em = oh1.astype(jnp.float32) + oh2.astype(jnp.float32)
    f = jnp.sum(em, axis=0, keepdims=True) / (N_TOK * 2.0)
    P = jnp.sum(probs, axis=0, keepdims=True) / N_TOK
    lbl_ref[...] = E * jnp.sum(f * P, axis=1, keepdims=True)


def _dispatch_body(te_hbm, tw_hbm, x_hbm,
                   pos_hbm, xs_hbm, ws_hbm, cnt_hbm,
                   e_all, pos_chunk, pos_half, wvals, wrow, idx_tok, xbuf,
                   cnt_buf, run_buf, sem):
    wid = lax.axis_index("s") * NC + lax.axis_index("c")
    base_p = wid * CHUNK
    my_first_vreg = base_p // L

    pltpu.sync_copy(te_hbm, e_all)

    lane = lax.iota(jnp.int32, L)
    zero_v = jnp.zeros((L,), jnp.int32)

    def count_step(i, carry):
        tot, bef = carry
        v = e_all[pl.ds(i * L, L)]
        pre = i < my_first_vreg
        new_tot = []
        new_bef = []
        for e in range(E):
            m = (v == e).astype(jnp.int32)
            new_tot.append(tot[e] + m)
            new_bef.append(bef[e] + jnp.where(pre, m, 0))
        return tuple(new_tot), tuple(new_bef)

    tot0 = tuple(zero_v for _ in range(E))
    tot, bef = lax.fori_loop(0, N_DISP // L, count_step, (tot0, tot0))
    total = zero_v
    before = zero_v
    for e in range(E):
        te_sum = plsc.cumsum(tot[e])[L - 1]
        be_sum = plsc.cumsum(bef[e])[L - 1]
        total = total + jnp.where(lane == e, te_sum, 0)
        before = before + jnp.where(lane == e, be_sum, 0)

    @pl.when(wid == 0)
    def _():
        cnt_buf[...] = total
        pltpu.sync_copy(cnt_buf, cnt_hbm)

    off_excl = plsc.cumsum(total) - total
    run_buf[...] = off_excl + before

    for i in range(CHUNK // L):
        v = e_all[pl.ds((my_first_vreg + i) * L, L)]
        start_here = plsc.load_gather(run_buf, [v])
        posv = zero_v
        hist = zero_v
        for e in range(E):
            m = (v == e)
            c = plsc.cumsum(m.astype(jnp.int32))
            posv = posv + jnp.where(m, c - 1, 0)
            hist = hist + jnp.where(lane == e, c[L - 1], 0)
        pos_chunk[pl.ds(i * L, L)] = start_here + posv
        run_buf[...] = run_buf[...] + hist

    pltpu.sync_copy(pos_chunk, pos_hbm.at[pl.ds(base_p, CHUNK)])

    pltpu.sync_copy(tw_hbm.at[pl.ds(base_p, CHUNK)], wvals)
    for c in range(CHUNK // L):
        wv = wvals[pl.ds(c * L, L)]
        r = c * L + lane
        plsc.store_scatter(wrow, [r, zero_v], wv)
    pltpu.async_copy(wrow, ws_hbm.at[pos_chunk], sem).wait()

    for half in range(2):
        hbase = base_p + half * (CHUNK // 2)
        for c in range(CHUNK // 2 // L):
            tok = lax.shift_right_logical(hbase + c * L + lane, 1)
            idx_tok[pl.ds(c * L, L)] = tok
            pos_half[pl.ds(c * L, L)] = pos_chunk[pl.ds(half * (CHUNK // 2)
                                                        + c * L, L)]
        pltpu.async_copy(x_hbm.at[idx_tok], xbuf, sem).wait()
        pltpu.async_copy(xbuf, xs_hbm.at[pos_half], sem).wait()


def _gmm_body(se_ref, st_ref, slo_ref, shi_ref, sfirst_ref,
              xs_ref, ws_ref, wsu_ref, wru_ref, wsd_ref, wrd_ref, out_ref):
    h_idx = pl.program_id(0)
    s_idx = pl.program_id(1)

    @pl.when((h_idx == 0) & (s_idx == 0))
    def _():
        out_ref[...] = jnp.zeros_like(out_ref)

    t = st_ref[s_idx]
    lo = slo_ref[s_idx]
    hi = shi_ref[s_idx]
    base = t * TM
    row = base + lax.broadcasted_iota(jnp.int32, (TM, 1), 0)
    msk = (row >= lo) & (row < hi)
    xm = jnp.where(msk, xs_ref[...], 0.0)
    we = (wsu_ref[...] + wru_ref[0]).astype(jnp.bfloat16)
    hblk = jnp.dot(xm.astype(jnp.bfloat16), we,
                   preferred_element_type=jnp.float32)
    a = jnp.square(jnp.maximum(hblk, 0.0))
    aw = (a * ws_ref[pl.ds(base, TM), 0:1]).astype(jnp.bfloat16)
    wd = (wsd_ref[...] + wrd_ref[0]).astype(jnp.bfloat16)
    y = lax.dot_general(aw, wd, (((1,), (1,)), ((), ())),
                        preferred_element_type=jnp.float32)
    cur = out_ref[pl.ds(base, TM), :]
    out_ref[pl.ds(base, TM), :] = cur + y


def _combine_body(pos_hbm, ysw_hbm, out_hbm,
                  posbuf, idx_even, idx_odd, ybuf, ybuf2, sem):
    wid = lax.axis_index("s") * NC + lax.axis_index("c")
    base_t = wid * (N_TOK // NW)
    lane = lax.iota(jnp.int32, L)

    pltpu.sync_copy(pos_hbm.at[pl.ds(base_t * 2, CHUNK)], posbuf)

    for it in range(2):
        for c in range(2):
            sbase = it * 64 + c * 2 * L
            ev = plsc.load_gather(posbuf, [sbase + 2 * lane])
            od = plsc.load_gather(posbuf, [sbase + 2 * lane + 1])
            idx_even[pl.ds(c * L, L)] = ev
            idx_odd[pl.ds(c * L, L)] = od
        pltpu.async_copy(ysw_hbm.at[idx_even], ybuf, sem).wait()
        pltpu.async_copy(ysw_hbm.at[idx_odd], ybuf2, sem).wait()

        def add_step(i, _):
            j = i // (D // L)
            cc = i % (D // L)
            ybuf[j, pl.ds(cc * L, L)] = (ybuf[j, pl.ds(cc * L, L)]
                                         + ybuf2[j, pl.ds(cc * L, L)])
            return 0

        lax.fori_loop(0, 32 * (D // L), add_step, 0)
        pltpu.sync_copy(ybuf, out_hbm.at[pl.ds(base_t + it * 32, 32)])


def _step_metadata(cnt):
    cnt = cnt[:E]
    off = jnp.concatenate([jnp.zeros((1,), jnp.int32),
                           jnp.cumsum(cnt, dtype=jnp.int32)])
    first_t = off[:E] // TM
    last_t = (off[1:] + TM - 1) // TM - 1
    n_e = jnp.where(cnt > 0, last_t - first_t + 1, 0)
    estart = jnp.concatenate([jnp.zeros((1,), jnp.int32),
                              jnp.cumsum(n_e, dtype=jnp.int32)])
    total = estart[E]
    s = jnp.arange(S_STEPS, dtype=jnp.int32)
    e_s = jnp.sum((s[:, None] >= estart[1:][None, :]).astype(jnp.int32),
                  axis=1)
    valid = s < total
    e_c = jnp.minimum(e_s, E - 1)
    tile = first_t[e_c] + (s - estart[e_c])
    tile = jnp.where(valid, tile, NMD - 1).astype(jnp.int32)
    lo = jnp.where(valid, jnp.maximum(off[e_c], tile * TM), 0)
    hi = jnp.where(valid, jnp.minimum(off[e_c + 1], tile * TM + TM), 0)
    tile_prev = jnp.concatenate([jnp.full((1,), -1, jnp.int32), tile[:-1]])
    sfirst = (valid & (tile != tile_prev)).astype(jnp.int32)
    return (e_c.astype(jnp.int32), tile, lo.astype(jnp.int32),
            hi.astype(jnp.int32), sfirst)


def kernel(x, W_router, W_shared_up, W_shared_down, W_routed_up, W_routed_down):
    B, T, Dx = x.shape
    N = B * T
    xf = x.reshape(N, Dx)

    te, tw, lbl, zl = pl.pallas_call(
        _router_body,
        out_shape=[
            jax.ShapeDtypeStruct((N, 2), jnp.int32),
            jax.ShapeDtypeStruct((N, 2), jnp.float32),
            jax.ShapeDtypeStruct((1, 1), jnp.float32),
            jax.ShapeDtypeStruct((1, 1), jnp.float32),
        ],
    )(xf, W_router)

    te_flat = te.reshape(N_DISP)
    tw_flat = tw.reshape(N_DISP)

    mesh = plsc.VectorSubcoreMesh(core_axis_name="c", subcore_axis_name="s",
                                  num_cores=NC)
    dispatch = functools.partial(
        pl.kernel,
        out_type=[
            jax.ShapeDtypeStruct((N_DISP,), jnp.int32),
            jax.ShapeDtypeStruct((N_DISP, D), jnp.float32),
            jax.ShapeDtypeStruct((N_DISP, 128), jnp.float32),
            jax.ShapeDtypeStruct((L,), jnp.int32),
        ],
        mesh=mesh,
        compiler_params=pltpu.CompilerParams(needs_layout_passes=False),
        scratch_types=[
            pltpu.VMEM((N_DISP,), jnp.int32),
            pltpu.VMEM((CHUNK,), jnp.int32),
            pltpu.VMEM((CHUNK // 2,), jnp.int32),
            pltpu.VMEM((CHUNK,), jnp.float32),
            pltpu.VMEM((CHUNK, 128), jnp.float32),
            pltpu.VMEM((CHUNK // 2,), jnp.int32),
            pltpu.VMEM((CHUNK // 2, D), jnp.float32),
            pltpu.VMEM((L,), jnp.int32),
            pltpu.VMEM((L,), jnp.int32),
            pltpu.SemaphoreType.DMA,
        ],
    )(_dispatch_body)
    pos, xs, ws, cnt = dispatch(te_flat, tw_flat, xf)

    se, st, slo, shi, sfirst = _step_metadata(cnt)

    ysw = pl.pallas_call(
        _gmm_body,
        grid_spec=pltpu.PrefetchScalarGridSpec(
            num_scalar_prefetch=5,
            grid=(NHD, S_STEPS),
            in_specs=[
                pl.BlockSpec((TM, D),
                             lambda h, s, se_, st_, *_: (st_[s], 0)),
                pl.BlockSpec((N_DISP, 128), lambda h, s, *_: (0, 0)),
                pl.BlockSpec((D, TH), lambda h, s, *_: (0, h)),
                pl.BlockSpec((1, D, TH),
                             lambda h, s, se_, *_: (se_[s], 0, h)),
                pl.BlockSpec((D, TH), lambda h, s, *_: (0, h)),
                pl.BlockSpec((1, D, TH),
                             lambda h, s, se_, *_: (se_[s], 0, h)),
            ],
            out_specs=pl.BlockSpec((N_DISP, D), lambda h, s, *_: (0, 0)),
        ),
        out_shape=jax.ShapeDtypeStruct((N_DISP, D), jnp.float32),
    )(se, st, slo, shi, sfirst, xs, ws, W_shared_up, W_routed_up,
      W_shared_down, W_routed_down)

    combine = functools.partial(
        pl.kernel,
        out_type=jax.ShapeDtypeStruct((N, D), jnp.float32),
        mesh=mesh,
        compiler_params=pltpu.CompilerParams(needs_layout_passes=False),
        scratch_types=[
            pltpu.VMEM((CHUNK,), jnp.int32),
            pltpu.VMEM((2 * L,), jnp.int32),
            pltpu.VMEM((2 * L,), jnp.int32),
            pltpu.VMEM((32, D), jnp.float32),
            pltpu.VMEM((32, D), jnp.float32),
            pltpu.SemaphoreType.DMA,
        ],
    )(_combine_body)
    out = combine(pos, ysw)

    return out.reshape(B, T, Dx), lbl.reshape(()), zl.reshape(())


# speedup vs baseline: 1.0754x; 1.0632x over previous
import functools

import jax
import jax.numpy as jnp
from jax import lax
from jax.experimental import pallas as pl
from jax.experimental.pallas import tpu as pltpu
from jax.experimental.pallas import tpu_sc as plsc

D = 1024
H = 4096
E = 8
N_TOK = 2048
N_DISP = N_TOK * 2

NC = 2
NS = 16
NW = NC * NS
CHUNK = N_DISP // NW
L = 16

TM = 256
TH = 1024
NMD = N_DISP // TM
NHD = H // TH
S_STEPS = NMD + E - 1


def _router_body(x_ref, wr_ref, te_ref, tw_ref, lbl_ref, zl_ref):
    x = x_ref[...]
    wr = wr_ref[...]
    logits = lax.dot_general(x, wr, (((1,), (1,)), ((), ())),
                             preferred_element_type=jnp.float32)
    mx = jnp.max(logits, axis=-1, keepdims=True)
    ex = jnp.exp(logits - mx)
    denom = jnp.sum(ex, axis=-1, keepdims=True)
    probs = ex / denom
    col = lax.broadcasted_iota(jnp.int32, probs.shape, 1)
    m1 = jnp.max(probs, axis=-1, keepdims=True)
    i1 = jnp.min(jnp.where(probs == m1, col, E), axis=-1, keepdims=True)
    oh1 = (col == i1)
    p2 = jnp.where(oh1, -1.0, probs)
    m2 = jnp.max(p2, axis=-1, keepdims=True)
    i2 = jnp.min(jnp.where(p2 == m2, col, E), axis=-1, keepdims=True)
    oh2 = (col == i2)
    wsum = m1 + m2
    te_ref[...] = jnp.concatenate([i1, i2], axis=1)
    tw_ref[...] = jnp.concatenate([m1 / wsum, m2 / wsum], axis=1)
    lse = jnp.log(denom) + mx
    zl_ref[...] = jnp.sum(jnp.square(lse), axis=0, keepdims=True) / N_TOK
    em = oh1.astype(jnp.float32) + oh2.astype(jnp.float32)
    f = jnp.sum(em, axis=0, keepdims=True) / (N_TOK * 2.0)
    P = jnp.sum(probs, axis=0, keepdims=True) / N_TOK
    lbl_ref[...] = E * jnp.sum(f * P, axis=1, keepdims=True)


def _dispatch_body(te_hbm, tw_hbm, x_hbm,
                   pos_hbm, xs_hbm, ws_hbm, cnt_hbm,
                   e_all, pos_chunk, pos_half, wvals, wrow, idx_tok, xbuf,
                   cnt_buf, run_buf, sem):
    wid = lax.axis_index("s") * NC + lax.axis_index("c")
    base_p = wid * CHUNK
    my_first_vreg = base_p // L

    pltpu.sync_copy(te_hbm, e_all)

    lane = lax.iota(jnp.int32, L)
    zero_v = jnp.zeros((L,), jnp.int32)

    def count_step(i, carry):
        tot, bef = carry
        v = e_all[pl.ds(i * L, L)]
        pre = i < my_first_vreg
        new_tot = []
        new_bef = []
        for e in range(E):
            m = (v == e).astype(jnp.int32)
            new_tot.append(tot[e] + m)
            new_bef.append(bef[e] + jnp.where(pre, m, 0))
        return tuple(new_tot), tuple(new_bef)

    tot0 = tuple(zero_v for _ in range(E))
    tot, bef = lax.fori_loop(0, N_DISP // L, count_step, (tot0, tot0))
    total = zero_v
    before = zero_v
    for e in range(E):
        te_sum = plsc.cumsum(tot[e])[L - 1]
        be_sum = plsc.cumsum(bef[e])[L - 1]
        total = total + jnp.where(lane == e, te_sum, 0)
        before = before + jnp.where(lane == e, be_sum, 0)

    @pl.when(wid == 0)
    def _():
        cnt_buf[...] = total
        pltpu.sync_copy(cnt_buf, cnt_hbm)

    off_excl = plsc.cumsum(total) - total
    run_buf[...] = off_excl + before

    for i in range(CHUNK // L):
        v = e_all[pl.ds((my_first_vreg + i) * L, L)]
        start_here = plsc.load_gather(run_buf, [v])
        posv = zero_v
        hist = zero_v
        for e in range(E):
            m = (v == e)
            c = plsc.cumsum(m.astype(jnp.int32))
            posv = posv + jnp.where(m, c - 1, 0)
            hist = hist + jnp.where(lane == e, c[L - 1], 0)
        pos_chunk[pl.ds(i * L, L)] = start_here + posv
        run_buf[...] = run_buf[...] + hist

    pltpu.sync_copy(pos_chunk, pos_hbm.at[pl.ds(base_p, CHUNK)])

    pltpu.sync_copy(tw_hbm.at[pl.ds(base_p, CHUNK)], wvals)
    for c in range(CHUNK // L):
        wv = wvals[pl.ds(c * L, L)]
        r = c * L + lane
        plsc.store_scatter(wrow, [r, zero_v], wv)
    pltpu.async_copy(wrow, ws_hbm.at[pos_chunk], sem).wait()

    for half in range(2):
        hbase = base_p + half * (CHUNK // 2)
        for c in range(CHUNK // 2 // L):
            tok = lax.shift_right_logical(hbase + c * L + lane, 1)
            idx_tok[pl.ds(c * L, L)] = tok
            pos_half[pl.ds(c * L, L)] = pos_chunk[pl.ds(half * (CHUNK // 2)
                                                        + c * L, L)]
        pltpu.async_copy(x_hbm.at[idx_tok], xbuf, sem).wait()
        pltpu.async_copy(xbuf, xs_hbm.at[pos_half], sem).wait()


def _gmm_body(se_ref, st_ref, slo_ref, shi_ref, sfirst_ref,
              xs_ref, ws_ref, wsu_ref, wru_ref, wsd_ref, wrd_ref, out_ref):
    h_idx = pl.program_id(0)
    s_idx = pl.program_id(1)

    @pl.when((h_idx == 0) & (s_idx == 0))
    def _():
        out_ref[...] = jnp.zeros_like(out_ref)

    t = st_ref[s_idx]
    lo = slo_ref[s_idx]
    hi = shi_ref[s_idx]
    base = t * TM
    row = base + lax.broadcasted_iota(jnp.int32, (TM, 1), 0)
    msk = (row >= lo) & (row < hi)
    xm = jnp.where(msk, xs_ref[...], 0.0)
    we = (wsu_ref[...] + wru_ref[0]).astype(jnp.bfloat16)
    hblk = jnp.dot(xm.astype(jnp.bfloat16), we,
                   preferred_element_type=jnp.float32)
    a = jnp.square(jnp.maximum(hblk, 0.0))
    aw = (a * ws_ref[pl.ds(base, TM), 0:1]).astype(jnp.bfloat16)
    wd = (wsd_ref[...] + wrd_ref[0]).astype(jnp.bfloat16)
    y = lax.dot_general(aw, wd, (((1,), (1,)), ((), ())),
                        preferred_element_type=jnp.float32)
    cur = out_ref[pl.ds(base, TM), :]
    out_ref[pl.ds(base, TM), :] = cur + y


def _combine_body(pos_hbm, ysw_hbm, out_hbm,
                  posbuf, idx_even, idx_odd, ybuf, ybuf2, sem):
    wid = lax.axis_index("s") * NC + lax.axis_index("c")
    base_t = wid * (N_TOK // NW)
    lane = lax.iota(jnp.int32, L)

    pltpu.sync_copy(pos_hbm.at[pl.ds(base_t * 2, CHUNK)], posbuf)

    for it in range(2):
        for c in range(2):
            sbase = it * 64 + c * 2 * L
            ev = plsc.load_gather(posbuf, [sbase + 2 * lane])
            od = plsc.load_gather(posbuf, [sbase + 2 * lane + 1])
            idx_even[pl.ds(c * L, L)] = ev
            idx_odd[pl.ds(c * L, L)] = od
        pltpu.async_copy(ysw_hbm.at[idx_even], ybuf, sem).wait()
        pltpu.async_copy(ysw_hbm.at[idx_odd], ybuf, sem, add=True).wait()
        pltpu.sync_copy(ybuf, out_hbm.at[pl.ds(base_t + it * 32, 32)])


def _step_metadata(cnt):
    cnt = cnt[:E]
    off = jnp.concatenate([jnp.zeros((1,), jnp.int32),
                           jnp.cumsum(cnt, dtype=jnp.int32)])
    first_t = off[:E] // TM
    last_t = (off[1:] + TM - 1) // TM - 1
    n_e = jnp.where(cnt > 0, last_t - first_t + 1, 0)
    estart = jnp.concatenate([jnp.zeros((1,), jnp.int32),
                              jnp.cumsum(n_e, dtype=jnp.int32)])
    total = estart[E]
    s = jnp.arange(S_STEPS, dtype=jnp.int32)
    e_s = jnp.sum((s[:, None] >= estart[1:][None, :]).astype(jnp.int32),
                  axis=1)
    valid = s < total
    e_c = jnp.minimum(e_s, E - 1)
    tile = first_t[e_c] + (s - estart[e_c])
    tile = jnp.where(valid, tile, NMD - 1).astype(jnp.int32)
    lo = jnp.where(valid, jnp.maximum(off[e_c], tile * TM), 0)
    hi = jnp.where(valid, jnp.minimum(off[e_c + 1], tile * TM + TM), 0)
    tile_prev = jnp.concatenate([jnp.full((1,), -1, jnp.int32), tile[:-1]])
    sfirst = (valid & (tile != tile_prev)).astype(jnp.int32)
    return (e_c.astype(jnp.int32), tile, lo.astype(jnp.int32),
            hi.astype(jnp.int32), sfirst)


def kernel(x, W_router, W_shared_up, W_shared_down, W_routed_up, W_routed_down):
    B, T, Dx = x.shape
    N = B * T
    xf = x.reshape(N, Dx)

    te, tw, lbl, zl = pl.pallas_call(
        _router_body,
        out_shape=[
            jax.ShapeDtypeStruct((N, 2), jnp.int32),
            jax.ShapeDtypeStruct((N, 2), jnp.float32),
            jax.ShapeDtypeStruct((1, 1), jnp.float32),
            jax.ShapeDtypeStruct((1, 1), jnp.float32),
        ],
    )(xf, W_router)

    te_flat = te.reshape(N_DISP)
    tw_flat = tw.reshape(N_DISP)

    mesh = plsc.VectorSubcoreMesh(core_axis_name="c", subcore_axis_name="s",
                                  num_cores=NC)
    dispatch = functools.partial(
        pl.kernel,
        out_type=[
            jax.ShapeDtypeStruct((N_DISP,), jnp.int32),
            jax.ShapeDtypeStruct((N_DISP, D), jnp.float32),
            jax.ShapeDtypeStruct((N_DISP, 128), jnp.float32),
            jax.ShapeDtypeStruct((L,), jnp.int32),
        ],
        mesh=mesh,
        compiler_params=pltpu.CompilerParams(needs_layout_passes=False),
        scratch_types=[
            pltpu.VMEM((N_DISP,), jnp.int32),
            pltpu.VMEM((CHUNK,), jnp.int32),
            pltpu.VMEM((CHUNK // 2,), jnp.int32),
            pltpu.VMEM((CHUNK,), jnp.float32),
            pltpu.VMEM((CHUNK, 128), jnp.float32),
            pltpu.VMEM((CHUNK // 2,), jnp.int32),
            pltpu.VMEM((CHUNK // 2, D), jnp.float32),
            pltpu.VMEM((L,), jnp.int32),
            pltpu.VMEM((L,), jnp.int32),
            pltpu.SemaphoreType.DMA,
        ],
    )(_dispatch_body)
    pos, xs, ws, cnt = dispatch(te_flat, tw_flat, xf)

    se, st, slo, shi, sfirst = _step_metadata(cnt)

    ysw = pl.pallas_call(
        _gmm_body,
        grid_spec=pltpu.PrefetchScalarGridSpec(
            num_scalar_prefetch=5,
            grid=(NHD, S_STEPS),
            in_specs=[
                pl.BlockSpec((TM, D),
                             lambda h, s, se_, st_, *_: (st_[s], 0)),
                pl.BlockSpec((N_DISP, 128), lambda h, s, *_: (0, 0)),
                pl.BlockSpec((D, TH), lambda h, s, *_: (0, h)),
                pl.BlockSpec((1, D, TH),
                             lambda h, s, se_, *_: (se_[s], 0, h)),
                pl.BlockSpec((D, TH), lambda h, s, *_: (0, h)),
                pl.BlockSpec((1, D, TH),
                             lambda h, s, se_, *_: (se_[s], 0, h)),
            ],
            out_specs=pl.BlockSpec((N_DISP, D), lambda h, s, *_: (0, 0)),
        ),
        out_shape=jax.ShapeDtypeStruct((N_DISP, D), jnp.float32),
    )(se, st, slo, shi, sfirst, xs, ws, W_shared_up, W_routed_up,
      W_shared_down, W_routed_down)

    combine = functools.partial(
        pl.kernel,
        out_type=jax.ShapeDtypeStruct((N, D), jnp.float32),
        mesh=mesh,
        compiler_params=pltpu.CompilerParams(needs_layout_passes=False),
        scratch_types=[
            pltpu.VMEM((CHUNK,), jnp.int32),
            pltpu.VMEM((2 * L,), jnp.int32),
            pltpu.VMEM((2 * L,), jnp.int32),
            pltpu.VMEM((32, D), jnp.float32),
            pltpu.VMEM((32, D), jnp.float32),
            pltpu.SemaphoreType.DMA,
        ],
    )(_combine_body)
    out = combine(pos, ysw)

    return out.reshape(B, T, Dx), lbl.reshape(()), zl.reshape(())
